# baseline (speedup 1.0000x reference)
import functools

import jax
import jax.numpy as jnp
from jax import lax
from jax.experimental import pallas as pl
from jax.experimental.pallas import tpu as pltpu, tpu_sc as plsc

DIM = 32
NC = 2
NS = 16
NW = NC * NS
CH = 4096
BB = 128


def _pack_body(ra_ref, rb_ref, ia_ref, ib_ref, o_ref):
    o_ref[...] = jnp.concatenate(
        [ra_ref[...].T, ia_ref[...].T, rb_ref[...].T, ib_ref[...].T], axis=1)


def _pack_tables(trp, tip):
    v = trp.shape[1]
    grid = (v + 2 * CH - 1) // (2 * CH)
    last = (v - 1) // CH

    def odd(i):
        return (0, jnp.minimum(2 * i + 1, last))

    return pl.pallas_call(
        _pack_body,
        grid=(grid,),
        in_specs=[
            pl.BlockSpec((DIM, CH), lambda i: (0, 2 * i)),
            pl.BlockSpec((DIM, CH), odd),
            pl.BlockSpec((DIM, CH), lambda i: (0, 2 * i)),
            pl.BlockSpec((DIM, CH), odd),
        ],
        out_specs=pl.BlockSpec((CH, 4 * DIM), lambda i: (i, 0)),
        out_shape=jax.ShapeDtypeStruct((grid * CH, 4 * DIM), jnp.float32),
    )(trp, trp, tip, tip)


@functools.partial(jax.jit, static_argnames=("n_l", "n_b"))
def _embed_call(ids_pad, table_r, table_i, n_l, n_b):
    n_lp = ids_pad.shape[0]
    n_bb = n_b // BB
    n_sb = (n_lp // 8) * n_bb
    k_per_w = n_sb // NW

    tab = _pack_tables(table_r.T, table_i.T)
    tab64 = tab.reshape(tab.shape[0] * 2, 2 * DIM)

    mesh = plsc.VectorSubcoreMesh(core_axis_name="c", subcore_axis_name="s")

    l_tail = max(n_l % 8, 1)
    m_full = (n_l - n_l % 8) * n_bb // NW
    m_total = m_full + ((n_l % 8) * n_bb) // NW
    assert m_total % 4 == 0

    @functools.partial(
        pl.kernel,
        out_type=jax.ShapeDtypeStruct((n_l, DIM, n_bb, 2, BB), jnp.float32),
        mesh=mesh,
        compiler_params=pltpu.CompilerParams(
            needs_layout_passes=False, use_tc_tiling_on_sc=False),
        scratch_types=[
            pltpu.VMEM((8, BB), jnp.int32),
            pltpu.VMEM((4, BB), jnp.int32),
            pltpu.VMEM((4, BB, 2 * DIM), jnp.float32),
            pltpu.VMEM((4, DIM, 2, BB), jnp.float32),
            pltpu.SemaphoreType.DMA,
            pltpu.SemaphoreType.DMA,
            pltpu.SemaphoreType.DMA,
            pltpu.SemaphoreType.DMA,
            pltpu.SemaphoreType.DMA,
            pltpu.SemaphoreType.DMA,
            pltpu.SemaphoreType.DMA,
            pltpu.SemaphoreType.DMA,
        ],
    )
    def k(ids_hbm, tab_hbm, out_hbm, idx_v, g_v, gbuf, tbuf,
          sem_g0, sem_g1, sem_g2, sem_g3, sem_w0, sem_w1, sem_w2, sem_w3):
        wid = lax.axis_index("s") * NC + lax.axis_index("c")
        lane = lax.iota(jnp.int32, 16)
        sem_g = (sem_g0, sem_g1, sem_g2, sem_g3)
        sem_w = (sem_w0, sem_w1, sem_w2, sem_w3)

        def coords(m):
            sbt = jnp.where(m < m_full, m // 8, m_full // 8 + (m - m_full) // l_tail)
            l8 = jnp.where(m < m_full, m % 8, (m - m_full) % l_tail)
            sb = sbt * NW + wid
            lo = (sb // n_bb) * 8
            bb = sb % n_bb
            return lo, l8, bb

        def prep(m, slot):
            lo, l8, bb = coords(m)

            @pl.when(l8 == 0)
            def _():
                pltpu.sync_copy(
                    ids_hbm.at[pl.ds(lo, 8), pl.ds(bb * BB, BB)], idx_v)

            for t in range(8):
                v = idx_v[l8, pl.ds(16 * t, 16)]
                g_v[slot, pl.ds(16 * t, 16)] = (
                    lax.shift_left(lax.shift_right_logical(v, 13), 13)
                    | lax.shift_left(v & (CH - 1), 1)
                    | (lax.shift_right_logical(v, 12) & 1))
            pltpu.async_copy(
                tab_hbm.at[g_v.at[slot]], gbuf.at[slot], sem_g[slot])

        def consume(m, slot):
            lo, l8, bb = coords(m)
            l = lo + l8
            @pl.when(m >= 4)
            def _():
                pltpu.make_async_copy(
                    tbuf.at[slot], out_hbm.at[0, :, 0, :, :], sem_w[slot]).wait()

            zero = jnp.zeros((16,), jnp.int32)
            for t in range(8):
                rows = lane + 16 * t

                @plsc.parallel_loop(0, DIM, unroll=4)
                def _(d):
                    c0 = zero + d
                    tbuf[slot, d, 0, pl.ds(16 * t, 16)] = plsc.load_gather(
                        gbuf.at[slot], [rows, c0])
                    tbuf[slot, d, 1, pl.ds(16 * t, 16)] = plsc.load_gather(
                        gbuf.at[slot], [rows, c0 + DIM])
            pltpu.async_copy(
                tbuf.at[slot], out_hbm.at[l, :, bb, :, :], sem_w[slot])

        def wait_g(slot):
            pltpu.make_async_copy(
                tab_hbm.at[g_v.at[slot]], gbuf.at[slot], sem_g[slot]).wait()

        prep(0, 0)
        prep(1, 1)
        prep(2, 2)

        def body4(j, _):
            for s in range(4):
                m = 4 * j + s
                wait_g(s)
                consume(m, s)

                @pl.when(m + 3 < m_total)
                def _():
                    prep(m + 3, (s + 3) % 4)

            return 0

        lax.fori_loop(0, m_total // 4, body4, 0)
        for slot in (0, 1, 2, 3):
            pltpu.make_async_copy(
                tbuf.at[slot], out_hbm.at[0, :, 0, :, :], sem_w[slot]).wait()

    return k(ids_pad, tab64)


def kernel(input_ids, table_r, table_i):
    b, l = input_ids.shape
    n_lp = ((l + 7) // 8) * 8
    idsp = input_ids.astype(jnp.int32).T
    ids_pad = jnp.pad(idsp, ((0, n_lp - l), (0, 0)))
    out3 = _embed_call(ids_pad, table_r, table_i, l, b)
    t = jnp.transpose(out3, (2, 4, 0, 1, 3))
    return t.reshape(b, l, DIM, 2)


# speedup vs baseline: 1.7812x; 1.7812x over previous
import functools

import jax
import jax.numpy as jnp
from jax import lax
from jax.experimental import pallas as pl
from jax.experimental.pallas import tpu as pltpu, tpu_sc as plsc

DIM = 32
NC = 2
NS = 16
NW = NC * NS
CH = 4096
BB = 128


def _pack_body(ra_ref, rb_ref, ia_ref, ib_ref, o_ref):
    o_ref[...] = jnp.concatenate(
        [ra_ref[...].T, ia_ref[...].T, rb_ref[...].T, ib_ref[...].T], axis=1)


def _pack_tables(trp, tip):
    v = trp.shape[1]
    grid = (v + 2 * CH - 1) // (2 * CH)
    last = (v - 1) // CH

    def odd(i):
        return (0, jnp.minimum(2 * i + 1, last))

    return pl.pallas_call(
        _pack_body,
        grid=(grid,),
        in_specs=[
            pl.BlockSpec((DIM, CH), lambda i: (0, 2 * i)),
            pl.BlockSpec((DIM, CH), odd),
            pl.BlockSpec((DIM, CH), lambda i: (0, 2 * i)),
            pl.BlockSpec((DIM, CH), odd),
        ],
        out_specs=pl.BlockSpec((CH, 4 * DIM), lambda i: (i, 0)),
        out_shape=jax.ShapeDtypeStruct((grid * CH, 4 * DIM), jnp.float32),
    )(trp, trp, tip, tip)


@functools.partial(jax.jit, static_argnames=("n_l", "n_b"))
def _embed_call(ids_pad, table_r, table_i, n_l, n_b):
    n_lp = ids_pad.shape[0]
    n_bb = n_b // BB
    n_sb = (n_lp // 8) * n_bb
    k_per_w = n_sb // NW

    tab = _pack_tables(table_r.T, table_i.T)
    tab64 = tab.reshape(tab.shape[0] * 2, 2 * DIM)

    mesh = plsc.VectorSubcoreMesh(core_axis_name="c", subcore_axis_name="s")

    l_tail = max(n_l % 8, 1)
    m_full = (n_l - n_l % 8) * n_bb // NW
    m_total = m_full + ((n_l % 8) * n_bb) // NW
    assert m_total % 4 == 0

    @functools.partial(
        pl.kernel,
        out_type=jax.ShapeDtypeStruct((n_l, DIM, n_bb, 2, BB), jnp.float32),
        mesh=mesh,
        compiler_params=pltpu.CompilerParams(
            needs_layout_passes=False, use_tc_tiling_on_sc=False),
        scratch_types=[
            pltpu.VMEM((8, BB), jnp.int32),
            pltpu.VMEM((4, BB), jnp.int32),
            pltpu.VMEM((4, BB, 2 * DIM), jnp.float32),
            pltpu.VMEM((4, DIM, 2, BB + 1), jnp.float32),
            pltpu.SemaphoreType.DMA,
            pltpu.SemaphoreType.DMA,
            pltpu.SemaphoreType.DMA,
            pltpu.SemaphoreType.DMA,
            pltpu.SemaphoreType.DMA,
            pltpu.SemaphoreType.DMA,
            pltpu.SemaphoreType.DMA,
            pltpu.SemaphoreType.DMA,
        ],
    )
    def k(ids_hbm, tab_hbm, out_hbm, idx_v, g_v, gbuf, tbuf,
          sem_g0, sem_g1, sem_g2, sem_g3, sem_w0, sem_w1, sem_w2, sem_w3):
        wid = lax.axis_index("s") * NC + lax.axis_index("c")
        lane = lax.iota(jnp.int32, 16)
        sem_g = (sem_g0, sem_g1, sem_g2, sem_g3)
        sem_w = (sem_w0, sem_w1, sem_w2, sem_w3)
        d_idx, e_idx = [], []
        for st in range(4):
            f = lane + 16 * st
            d_idx.append(f % DIM)
            e_idx.append(f // DIM)

        def coords(m):
            sbt = jnp.where(m < m_full, m // 8, m_full // 8 + (m - m_full) // l_tail)
            l8 = jnp.where(m < m_full, m % 8, (m - m_full) % l_tail)
            sb = sbt * NW + wid
            lo = (sb // n_bb) * 8
            bb = sb % n_bb
            return lo, l8, bb

        def prep(m, slot):
            lo, l8, bb = coords(m)

            @pl.when(l8 == 0)
            def _():
                pltpu.sync_copy(
                    ids_hbm.at[pl.ds(lo, 8), pl.ds(bb * BB, BB)], idx_v)

            for t in range(8):
                v = idx_v[l8, pl.ds(16 * t, 16)]
                g_v[slot, pl.ds(16 * t, 16)] = (
                    lax.shift_left(lax.shift_right_logical(v, 13), 13)
                    | lax.shift_left(v & (CH - 1), 1)
                    | (lax.shift_right_logical(v, 12) & 1))
            pltpu.async_copy(
                tab_hbm.at[g_v.at[slot]], gbuf.at[slot], sem_g[slot])

        def consume(m, slot):
            lo, l8, bb = coords(m)
            l = lo + l8
            @pl.when(m >= 4)
            def _():
                pltpu.make_async_copy(
                    tbuf.at[slot, :, :, pl.ds(0, BB)],
                    out_hbm.at[0, :, 0, :, :], sem_w[slot]).wait()

            @plsc.parallel_loop(0, BB, unroll=2)
            def _(b):
                bs = jnp.zeros((16,), jnp.int32) + b
                for st in range(4):
                    val = gbuf[slot, b, pl.ds(16 * st, 16)]
                    plsc.store_scatter(
                        tbuf.at[slot], [d_idx[st], e_idx[st], bs], val)
            pltpu.async_copy(
                tbuf.at[slot, :, :, pl.ds(0, BB)],
                out_hbm.at[l, :, bb, :, :], sem_w[slot])

        def wait_g(slot):
            pltpu.make_async_copy(
                tab_hbm.at[g_v.at[slot]], gbuf.at[slot], sem_g[slot]).wait()

        prep(0, 0)
        prep(1, 1)
        prep(2, 2)

        def body4(j, _):
            for s in range(4):
                m = 4 * j + s
                wait_g(s)
                consume(m, s)

                @pl.when(m + 3 < m_total)
                def _():
                    prep(m + 3, (s + 3) % 4)

            return 0

        lax.fori_loop(0, m_total // 4, body4, 0)
        for slot in (0, 1, 2, 3):
            pltpu.make_async_copy(
                tbuf.at[slot, :, :, pl.ds(0, BB)],
                out_hbm.at[0, :, 0, :, :], sem_w[slot]).wait()

    return k(ids_pad, tab64)


def kernel(input_ids, table_r, table_i):
    b, l = input_ids.shape
    n_lp = ((l + 7) // 8) * 8
    idsp = input_ids.astype(jnp.int32).T
    ids_pad = jnp.pad(idsp, ((0, n_lp - l), (0, 0)))
    out3 = _embed_call(ids_pad, table_r, table_i, l, b)
    t = jnp.transpose(out3, (2, 4, 0, 1, 3))
    return t.reshape(b, l, DIM, 2)
